# baseline (speedup 1.0000x reference)
import functools

import jax
import jax.numpy as jnp
from jax import lax
from jax.experimental import pallas as pl
from jax.experimental.pallas import tpu as pltpu
from jax.experimental.pallas import tpu_sc as plsc

POSE_NUM = 100000
EMBED_DIM = 6
BATCH = 4096

_info = plsc.get_sparse_core_info()
_NC = _info.num_cores
_NS = _info.num_subcores
_NW = _NC * _NS
_BPW = BATCH // _NW

_mesh = plsc.VectorSubcoreMesh(core_axis_name="c", subcore_axis_name="s")


@functools.partial(
    pl.kernel,
    mesh=_mesh,
    out_type=jax.ShapeDtypeStruct((EMBED_DIM, BATCH), jnp.float32),
    scratch_types=[
        pltpu.VMEM((_BPW,), jnp.int32),
        pltpu.VMEM((EMBED_DIM, _BPW), jnp.float32),
        pltpu.SemaphoreType.DMA,
    ],
)
def _sc_gather(idx_hbm, flat_hbm, out_hbm, idx_v, col6_v, sem):
    wid = lax.axis_index("s") * _NC + lax.axis_index("c")
    base = wid * _BPW
    pltpu.sync_copy(idx_hbm.at[pl.ds(base, _BPW)], idx_v)
    copies = [
        pltpu.async_copy(
            flat_hbm.at[pl.ds(j * POSE_NUM, POSE_NUM)].at[idx_v],
            col6_v.at[j],
            sem,
        )
        for j in range(EMBED_DIM)
    ]
    for cp in copies:
        cp.wait()
    pltpu.sync_copy(col6_v, out_hbm.at[:, pl.ds(base, _BPW)])


def kernel(indices, params):
    flat = jnp.reshape(params.T, (POSE_NUM * EMBED_DIM,))
    out_t = _sc_gather(indices.astype(jnp.int32), flat)
    return out_t.T
